# baseline (speedup 1.0000x reference)
import functools
import jax
import jax.numpy as jnp
from jax import lax
from jax.experimental import pallas as pl
from jax.experimental.pallas import tpu as pltpu
from jax.experimental.pallas import tpu_sc as plsc

_B, _S, _D = 4096, 50, 64
_N = _B * _S
_NW = 32
_PER_W = _N // _NW
_CHUNK = 128
_NCH = _PER_W // _CHUNK
_SCALE = 8.0
_LANES = 16


def _body(tok_hbm, w_hbm, out_hbm, idx_v, buf, gsem):
    c = lax.axis_index("c")
    s = lax.axis_index("s")
    wid = s * 2 + c
    pltpu.sync_copy(tok_hbm.at[wid], idx_v)

    def chunk_body(cg, _):
        pltpu.async_copy(w_hbm.at[idx_v.at[cg]], buf, gsem).wait()

        def row_body(r, _):
            for j in range(_D // _LANES):
                sl = pl.ds(j * _LANES, _LANES)
                buf[r, sl] = buf[r, sl] * _SCALE
            return 0

        lax.fori_loop(0, _CHUNK, row_body, 0, unroll=4)
        base = wid * _PER_W + cg * _CHUNK
        pltpu.sync_copy(buf, out_hbm.at[pl.ds(base, _CHUNK)])
        return 0

    lax.fori_loop(0, _NCH, chunk_body, 0)


_launch = functools.partial(
    pl.kernel,
    out_type=jax.ShapeDtypeStruct((_N, _D), jnp.float32),
    mesh=plsc.VectorSubcoreMesh(core_axis_name="c", subcore_axis_name="s"),
    scratch_types=[
        pltpu.VMEM((_NCH, _CHUNK), jnp.int32),
        pltpu.VMEM((_CHUNK, _D), jnp.float32),
        pltpu.SemaphoreType.DMA,
    ],
    compiler_params=pltpu.CompilerParams(use_tc_tiling_on_sc=False),
)(_body)


def kernel(tokens, W):
    tok = tokens.reshape(_NW, _NCH, _CHUNK)
    out = _launch(tok, W)
    return out.reshape(_B, _S, _D)
